# baseline (speedup 1.0000x reference)
import functools

import jax
import jax.numpy as jnp
from jax import lax
from jax.experimental import pallas as pl
from jax.experimental.pallas import tpu as pltpu
from jax.experimental.pallas import tpu_sc as plsc

N_CUR = 50
N_HIST = 200
D = 10000
L = 16
NC = 2
NS = 16
NW = NC * NS
W_PAD = 208
HALF = 104
N_CHUNKS = W_PAD // L
ROWS_PER_TILE = (N_CUR + NW - 1) // NW


SLAB = 8
RPS = 16
N_STEP = (N_CUR + RPS - 1) // RPS
CUR_TC = N_STEP * RPS
D_PAD = 10112


def _row_gather_body(cur_ref, *refs):
    slabs, rows_ref = refs[:RPS], refs[RPS]
    g = pl.program_id(0)
    for j in range(RPS):
        r = cur_ref[jnp.minimum(g * RPS + j, N_CUR - 1)] % SLAB
        rows_ref[pl.ds(j * D_PAD, D)] = slabs[j][pl.ds(r, 1), :][0]


def _slab_spec(j):
    return pl.BlockSpec(
        (SLAB, D),
        lambda g, cur, j=j: (cur[jnp.minimum(g * RPS + j, N_CUR - 1)] // SLAB, 0),
    )


_gather_rows_tc = pl.pallas_call(
    _row_gather_body,
    grid_spec=pltpu.PrefetchScalarGridSpec(
        num_scalar_prefetch=1,
        grid=(N_STEP,),
        in_specs=[_slab_spec(j) for j in range(RPS)],
        out_specs=pl.BlockSpec((RPS * D_PAD,), lambda g, cur: (g,)),
    ),
    out_shape=jax.ShapeDtypeStruct((CUR_TC * D_PAD,), jnp.float32),
)


def _lane_reduce(v, op):
    vals = [v[k] for k in range(L)]
    while len(vals) > 1:
        vals = [op(vals[k], vals[k + 1]) for k in range(0, len(vals), 2)]
    return vals[0]


def _do_row(i, hist_v, idx_v, vals_v, out_v, sem, rows_hbm, out_hbm):
    base = i * D_PAD
    lanes = lax.broadcasted_iota(jnp.int32, (L,), 0)
    tail_mask = lanes < (N_HIST - (N_CHUNKS - 1) * L)
    for c in range(N_CHUNKS):
        idx = hist_v[pl.ds(c * L, L)] + base
        if c == N_CHUNKS - 1:
            idx = jnp.where(tail_mask, idx, base)
        idx_v[pl.ds(c * L, L)] = idx
    cp0 = pltpu.async_copy(
        rows_hbm.at[idx_v.at[pl.ds(0, HALF)]], vals_v.at[pl.ds(0, HALF)], sem
    )
    cp1 = pltpu.async_copy(
        rows_hbm.at[idx_v.at[pl.ds(HALF, HALF)]], vals_v.at[pl.ds(HALF, HALF)], sem
    )
    cp0.wait()
    cp1.wait()
    neg_inf = jnp.float32(-jnp.inf)
    vs = []
    mx_lane = None
    for c in range(N_CHUNKS):
        g = vals_v[pl.ds(c * L, L)]
        v = 1.0 / g
        if c == N_CHUNKS - 1:
            v = jnp.where(tail_mask, v, neg_inf)
        vs.append(v)
        mx_lane = v if mx_lane is None else jnp.maximum(mx_lane, v)
    m = _lane_reduce(mx_lane, jnp.maximum)
    es = []
    s_lane = None
    for v in vs:
        e = jnp.exp(v - m)
        es.append(e)
        s_lane = e if s_lane is None else s_lane + e
    s = _lane_reduce(s_lane, lambda a, b: a + b)
    for c, e in enumerate(es):
        out_v[pl.ds(c * L, L)] = e / s
    pltpu.sync_copy(out_v.at[pl.ds(0, N_HIST)], out_hbm.at[pl.ds(i * N_HIST, N_HIST)])


@functools.partial(
    pl.kernel,
    mesh=plsc.VectorSubcoreMesh(core_axis_name="c", subcore_axis_name="s"),
    out_type=jax.ShapeDtypeStruct((N_CUR * N_HIST,), jnp.float32),
    scratch_types=[
        pltpu.VMEM((W_PAD,), jnp.int32),
        pltpu.VMEM((W_PAD,), jnp.int32),
        pltpu.VMEM((W_PAD,), jnp.float32),
        pltpu.VMEM((W_PAD,), jnp.float32),
        pltpu.SemaphoreType.DMA,
    ],
)
def _attn_loc_sc(
    hist_hbm, rows_hbm, out_hbm,
    hist_v, idx_v, vals_v, out_v, sem,
):
    w = lax.axis_index("s") * NC + lax.axis_index("c")
    pltpu.sync_copy(hist_hbm, hist_v.at[pl.ds(0, N_HIST)])
    for p in range(ROWS_PER_TILE):
        i = w + NW * p
        if (p + 1) * NW <= N_CUR:
            _do_row(i, hist_v, idx_v, vals_v, out_v, sem, rows_hbm, out_hbm)
        else:
            @pl.when(i < N_CUR)
            def _():
                _do_row(i, hist_v, idx_v, vals_v, out_v, sem, rows_hbm, out_hbm)


def kernel(history, current, poi_distance_matrix):
    hist = history.astype(jnp.int32)
    cur = current.astype(jnp.int32)
    rows_flat = _gather_rows_tc(cur, *([poi_distance_matrix] * RPS))
    return _attn_loc_sc(hist, rows_flat).reshape(N_CUR, N_HIST)


# speedup vs baseline: 1.0174x; 1.0174x over previous
import functools

import jax
import jax.numpy as jnp
from jax import lax
from jax.experimental import pallas as pl
from jax.experimental.pallas import tpu as pltpu
from jax.experimental.pallas import tpu_sc as plsc

N_CUR = 50
N_HIST = 200
D = 10000
L = 16
NC = 2
NS = 16
NW = NC * NS
W_PAD = 208
HALF = 104
N_CHUNKS = W_PAD // L
ROWS_PER_TILE = (N_CUR + NW - 1) // NW


SLAB = 8
RPS = 8
N_STEP = (N_CUR + RPS - 1) // RPS
CUR_TC = N_STEP * RPS
D_PAD = 10112


def _row_gather_body(cur_ref, *refs):
    slabs, rows_ref = refs[:RPS], refs[RPS]
    g = pl.program_id(0)
    for j in range(RPS):
        r = cur_ref[jnp.minimum(g * RPS + j, N_CUR - 1)] % SLAB
        rows_ref[pl.ds(j * D_PAD, D)] = slabs[j][pl.ds(r, 1), :][0]


def _slab_spec(j):
    return pl.BlockSpec(
        (SLAB, D),
        lambda g, cur, j=j: (cur[jnp.minimum(g * RPS + j, N_CUR - 1)] // SLAB, 0),
    )


_gather_rows_tc = pl.pallas_call(
    _row_gather_body,
    grid_spec=pltpu.PrefetchScalarGridSpec(
        num_scalar_prefetch=1,
        grid=(N_STEP,),
        in_specs=[_slab_spec(j) for j in range(RPS)],
        out_specs=pl.BlockSpec((RPS * D_PAD,), lambda g, cur: (g,)),
    ),
    out_shape=jax.ShapeDtypeStruct((CUR_TC * D_PAD,), jnp.float32),
)


def _lane_reduce(v, op):
    vals = [v[k] for k in range(L)]
    while len(vals) > 1:
        vals = [op(vals[k], vals[k + 1]) for k in range(0, len(vals), 2)]
    return vals[0]


def _do_row(i, hist_v, idx_v, vals_v, out_v, sem, rows_hbm, out_hbm):
    base = i * D_PAD
    lanes = lax.broadcasted_iota(jnp.int32, (L,), 0)
    tail_mask = lanes < (N_HIST - (N_CHUNKS - 1) * L)
    for c in range(N_CHUNKS):
        idx = hist_v[pl.ds(c * L, L)] + base
        if c == N_CHUNKS - 1:
            idx = jnp.where(tail_mask, idx, base)
        idx_v[pl.ds(c * L, L)] = idx
    cp0 = pltpu.async_copy(
        rows_hbm.at[idx_v.at[pl.ds(0, HALF)]], vals_v.at[pl.ds(0, HALF)], sem
    )
    cp1 = pltpu.async_copy(
        rows_hbm.at[idx_v.at[pl.ds(HALF, HALF)]], vals_v.at[pl.ds(HALF, HALF)], sem
    )
    cp0.wait()
    cp1.wait()
    neg_inf = jnp.float32(-jnp.inf)
    vs = []
    mx_lane = None
    for c in range(N_CHUNKS):
        g = vals_v[pl.ds(c * L, L)]
        v = 1.0 / g
        if c == N_CHUNKS - 1:
            v = jnp.where(tail_mask, v, neg_inf)
        vs.append(v)
        mx_lane = v if mx_lane is None else jnp.maximum(mx_lane, v)
    m = _lane_reduce(mx_lane, jnp.maximum)
    es = []
    s_lane = None
    for v in vs:
        e = jnp.exp(v - m)
        es.append(e)
        s_lane = e if s_lane is None else s_lane + e
    s = _lane_reduce(s_lane, lambda a, b: a + b)
    for c, e in enumerate(es):
        out_v[pl.ds(c * L, L)] = e / s
    pltpu.sync_copy(out_v.at[pl.ds(0, N_HIST)], out_hbm.at[pl.ds(i * N_HIST, N_HIST)])


@functools.partial(
    pl.kernel,
    mesh=plsc.VectorSubcoreMesh(core_axis_name="c", subcore_axis_name="s"),
    out_type=jax.ShapeDtypeStruct((N_CUR * N_HIST,), jnp.float32),
    scratch_types=[
        pltpu.VMEM((W_PAD,), jnp.int32),
        pltpu.VMEM((W_PAD,), jnp.int32),
        pltpu.VMEM((W_PAD,), jnp.float32),
        pltpu.VMEM((W_PAD,), jnp.float32),
        pltpu.SemaphoreType.DMA,
    ],
)
def _attn_loc_sc(
    hist_hbm, rows_hbm, out_hbm,
    hist_v, idx_v, vals_v, out_v, sem,
):
    w = lax.axis_index("s") * NC + lax.axis_index("c")
    pltpu.sync_copy(hist_hbm, hist_v.at[pl.ds(0, N_HIST)])
    for p in range(ROWS_PER_TILE):
        i = w + NW * p
        if (p + 1) * NW <= N_CUR:
            _do_row(i, hist_v, idx_v, vals_v, out_v, sem, rows_hbm, out_hbm)
        else:
            @pl.when(i < N_CUR)
            def _():
                _do_row(i, hist_v, idx_v, vals_v, out_v, sem, rows_hbm, out_hbm)


def kernel(history, current, poi_distance_matrix):
    hist = history.astype(jnp.int32)
    cur = current.astype(jnp.int32)
    rows_flat = _gather_rows_tc(cur, *([poi_distance_matrix] * RPS))
    return _attn_loc_sc(hist, rows_flat).reshape(N_CUR, N_HIST)


# speedup vs baseline: 1.0478x; 1.0299x over previous
import functools

import jax
import jax.numpy as jnp
from jax import lax
from jax.experimental import pallas as pl
from jax.experimental.pallas import tpu as pltpu
from jax.experimental.pallas import tpu_sc as plsc

N_CUR = 50
N_HIST = 200
D = 10000
L = 16
NC = 2
NS = 16
NW = NC * NS
W_PAD = 208
HALF = 104
N_CHUNKS = W_PAD // L
ROWS_PER_TILE = (N_CUR + NW - 1) // NW


SLAB = 8
RPS = 8
N_STEP = (N_CUR + RPS - 1) // RPS
CUR_TC = N_STEP * RPS
D_PAD = 10112


def _row_gather_body(cur_ref, *refs):
    slabs, rows_ref = refs[:RPS], refs[RPS]
    g = pl.program_id(0)
    for j in range(RPS):
        r = cur_ref[jnp.minimum(g * RPS + j, N_CUR - 1)] % SLAB
        rows_ref[pl.ds(j * D_PAD, D)] = slabs[j][pl.ds(r, 1), :][0]


def _slab_spec(j):
    return pl.BlockSpec(
        (SLAB, D),
        lambda g, cur, j=j: (cur[jnp.minimum(g * RPS + j, N_CUR - 1)] // SLAB, 0),
    )


_gather_rows_tc = pl.pallas_call(
    _row_gather_body,
    grid_spec=pltpu.PrefetchScalarGridSpec(
        num_scalar_prefetch=1,
        grid=(N_STEP,),
        in_specs=[_slab_spec(j) for j in range(RPS)],
        out_specs=pl.BlockSpec((RPS * D_PAD,), lambda g, cur: (g,)),
    ),
    out_shape=jax.ShapeDtypeStruct((CUR_TC * D_PAD,), jnp.float32),
)


def _lane_reduce(v, op):
    vals = [v[k] for k in range(L)]
    while len(vals) > 1:
        vals = [op(vals[k], vals[k + 1]) for k in range(0, len(vals), 2)]
    return vals[0]


def _issue_row(i, slot, hist_v, idx_v, vals_v, sem, rows_hbm):
    base = i * D_PAD
    off = slot * W_PAD
    lanes = lax.broadcasted_iota(jnp.int32, (L,), 0)
    tail_mask = lanes < (N_HIST - (N_CHUNKS - 1) * L)
    for c in range(N_CHUNKS):
        idx = hist_v[pl.ds(c * L, L)] + base
        if c == N_CHUNKS - 1:
            idx = jnp.where(tail_mask, idx, base)
        idx_v[pl.ds(off + c * L, L)] = idx
    cp0 = pltpu.async_copy(
        rows_hbm.at[idx_v.at[pl.ds(off, HALF)]],
        vals_v.at[pl.ds(off, HALF)], sem,
    )
    cp1 = pltpu.async_copy(
        rows_hbm.at[idx_v.at[pl.ds(off + HALF, HALF)]],
        vals_v.at[pl.ds(off + HALF, HALF)], sem,
    )
    return cp0, cp1


def _softmax_store(i, slot, vals_v, out_v, out_hbm):
    off = slot * W_PAD
    lanes = lax.broadcasted_iota(jnp.int32, (L,), 0)
    tail_mask = lanes < (N_HIST - (N_CHUNKS - 1) * L)
    neg_inf = jnp.float32(-jnp.inf)
    vs = []
    mx_lane = None
    for c in range(N_CHUNKS):
        g = vals_v[pl.ds(off + c * L, L)]
        v = 1.0 / g
        if c == N_CHUNKS - 1:
            v = jnp.where(tail_mask, v, neg_inf)
        vs.append(v)
        mx_lane = v if mx_lane is None else jnp.maximum(mx_lane, v)
    m = _lane_reduce(mx_lane, jnp.maximum)
    es = []
    s_lane = None
    for v in vs:
        e = jnp.exp(v - m)
        es.append(e)
        s_lane = e if s_lane is None else s_lane + e
    s = _lane_reduce(s_lane, lambda a, b: a + b)
    for c, e in enumerate(es):
        out_v[pl.ds(c * L, L)] = e / s
    pltpu.sync_copy(out_v.at[pl.ds(0, N_HIST)], out_hbm.at[pl.ds(i * N_HIST, N_HIST)])


@functools.partial(
    pl.kernel,
    mesh=plsc.VectorSubcoreMesh(core_axis_name="c", subcore_axis_name="s"),
    out_type=jax.ShapeDtypeStruct((N_CUR * N_HIST,), jnp.float32),
    scratch_types=[
        pltpu.VMEM((W_PAD,), jnp.int32),
        pltpu.VMEM((2 * W_PAD,), jnp.int32),
        pltpu.VMEM((2 * W_PAD,), jnp.float32),
        pltpu.VMEM((W_PAD,), jnp.float32),
        pltpu.SemaphoreType.DMA,
        pltpu.SemaphoreType.DMA,
    ],
)
def _attn_loc_sc(
    hist_hbm, rows_hbm, out_hbm,
    hist_v, idx_v, vals_v, out_v, sem0, sem1,
):
    w = lax.axis_index("s") * NC + lax.axis_index("c")
    pltpu.sync_copy(hist_hbm, hist_v.at[pl.ds(0, N_HIST)])
    i0 = w
    i1 = w + NW
    c00, c01 = _issue_row(i0, 0, hist_v, idx_v, vals_v, sem0, rows_hbm)
    cps1 = []

    @pl.when(i1 < N_CUR)
    def _():
        cps1.extend(_issue_row(i1, 1, hist_v, idx_v, vals_v, sem1, rows_hbm))

    c00.wait()
    c01.wait()
    _softmax_store(i0, 0, vals_v, out_v, out_hbm)

    @pl.when(i1 < N_CUR)
    def _():
        cps1[0].wait()
        cps1[1].wait()
        _softmax_store(i1, 1, vals_v, out_v, out_hbm)


def kernel(history, current, poi_distance_matrix):
    hist = history.astype(jnp.int32)
    cur = current.astype(jnp.int32)
    rows_flat = _gather_rows_tc(cur, *([poi_distance_matrix] * RPS))
    return _attn_loc_sc(hist, rows_flat).reshape(N_CUR, N_HIST)
